# baseline (speedup 1.0000x reference)
import functools

import jax
import jax.numpy as jnp
from jax import lax
from jax.experimental import pallas as pl
from jax.experimental.pallas import tpu as pltpu
from jax.experimental.pallas import tpu_sc as plsc

VOCAB = 100000
EMBED_DIM = 64
BATCH = 1024

_SC_INFO = plsc.get_sparse_core_info()
_NC = _SC_INFO.num_cores
_NS = _SC_INFO.num_subcores
_NW = _NC * _NS
_B_PER_W = BATCH // _NW

_V_BLK = 512


def _make_gather():
  mesh = plsc.VectorSubcoreMesh(core_axis_name="c", subcore_axis_name="s")

  @functools.partial(
      pl.kernel,
      mesh=mesh,
      out_type=jax.ShapeDtypeStruct((BATCH, EMBED_DIM), jnp.float32),
      scratch_types=[
          pltpu.VMEM((_B_PER_W,), jnp.int32),
          pltpu.VMEM((_B_PER_W, EMBED_DIM), jnp.float32),
          pltpu.SemaphoreType.DMA,
      ],
      compiler_params=pltpu.CompilerParams(use_tc_tiling_on_sc=False),
  )
  def gather_kernel(table_hbm, idx_hbm, out_hbm, idx_v, rows_v, sem):
    wid = lax.axis_index("s") * _NC + lax.axis_index("c")
    base = wid * _B_PER_W
    pltpu.sync_copy(idx_hbm.at[pl.ds(base, _B_PER_W)], idx_v)
    pltpu.async_copy(table_hbm.at[idx_v], rows_v, sem).wait()
    pltpu.sync_copy(rows_v, out_hbm.at[pl.ds(base, _B_PER_W)])

  return gather_kernel


_gather = _make_gather()


def _matmul_body(e_ref, w_ref, b_ref, o_ref):
  o_ref[...] = (
      lax.dot_general(
          e_ref[...],
          w_ref[...],
          (((1,), (1,)), ((), ())),
          preferred_element_type=jnp.float32,
      )
      + b_ref[...]
  )


@jax.jit
def kernel(inputs, embeddings, W, b):
  embeds = _gather(embeddings, inputs.astype(jnp.int32))
  n_blk = pl.cdiv(VOCAB, _V_BLK)
  out = pl.pallas_call(
      _matmul_body,
      grid=(n_blk,),
      in_specs=[
          pl.BlockSpec((BATCH, EMBED_DIM), lambda i: (0, 0)),
          pl.BlockSpec((_V_BLK, EMBED_DIM), lambda i: (i, 0)),
          pl.BlockSpec((1, _V_BLK), lambda i: (0, i)),
      ],
      out_specs=pl.BlockSpec((BATCH, _V_BLK), lambda i: (0, i)),
      out_shape=jax.ShapeDtypeStruct((BATCH, VOCAB), jnp.float32),
  )(embeds, W, b.reshape(1, VOCAB))
  return out


# speedup vs baseline: 1.1326x; 1.1326x over previous
import functools

import jax
import jax.numpy as jnp
from jax import lax
from jax.experimental import pallas as pl
from jax.experimental.pallas import tpu as pltpu
from jax.experimental.pallas import tpu_sc as plsc

VOCAB = 100000
EMBED_DIM = 64
BATCH = 1024

_SC_INFO = plsc.get_sparse_core_info()
_NC = _SC_INFO.num_cores
_NS = _SC_INFO.num_subcores
_NW = _NC * _NS
_B_PER_W = BATCH // _NW

_V_BLK = 2048
_N_FULL = VOCAB // _V_BLK
_TAIL = VOCAB - _N_FULL * _V_BLK
_N_BLK = _N_FULL + 1
_NBUF = 3
_SPLIT = 4
_RB = BATCH // _SPLIT


def _make_gather():
  mesh = plsc.VectorSubcoreMesh(core_axis_name="c", subcore_axis_name="s")

  @functools.partial(
      pl.kernel,
      mesh=mesh,
      out_type=jax.ShapeDtypeStruct((BATCH, EMBED_DIM), jnp.float32),
      scratch_types=[
          pltpu.VMEM((_B_PER_W,), jnp.int32),
          pltpu.VMEM((_B_PER_W, EMBED_DIM), jnp.float32),
          pltpu.SemaphoreType.DMA,
      ],
      compiler_params=pltpu.CompilerParams(use_tc_tiling_on_sc=False),
  )
  def gather_kernel(table_hbm, idx_hbm, out_hbm, idx_v, rows_v, sem):
    wid = lax.axis_index("s") * _NC + lax.axis_index("c")
    base = wid * _B_PER_W
    pltpu.sync_copy(idx_hbm.at[pl.ds(base, _B_PER_W)], idx_v)
    pltpu.async_copy(table_hbm.at[idx_v], rows_v, sem).wait()
    pltpu.sync_copy(rows_v, out_hbm.at[pl.ds(base, _B_PER_W)])

  return gather_kernel


_gather = _make_gather()


def _full_copy(out_hbm, acc, sems, blk, slot, s):
  return pltpu.make_async_copy(
      acc.at[slot, pl.ds(s * _RB, _RB), :],
      out_hbm.at[pl.ds(s * _RB, _RB), pl.ds(blk * _V_BLK, _V_BLK)],
      sems.at[slot, s],
  )


def _matmul_body(e_ref, w_ref, b_ref, out_hbm, acc, tacc, sems, tsems):
  i = pl.program_id(0)

  @pl.when(i >= _NBUF)
  def _wait_slot():
    for k in range(_NBUF):
      @pl.when(i % _NBUF == k)
      def _():
        for s in range(_SPLIT):
          _full_copy(out_hbm, acc, sems, i - _NBUF, k, s).wait()

  e = e_ref[...]

  @pl.when(i < _N_FULL)
  def _full_block():
    res = (
        lax.dot_general(
            e, w_ref[...], (((1,), (1,)), ((), ())),
            preferred_element_type=jnp.float32,
        )
        + b_ref[...]
    )
    for k in range(_NBUF):
      @pl.when(i % _NBUF == k)
      def _():
        acc[k] = res
        for s in range(_SPLIT):
          _full_copy(out_hbm, acc, sems, i, k, s).start()

  @pl.when(i == _N_FULL)
  def _tail_block():
    tacc[...] = (
        lax.dot_general(
            e, w_ref[: _TAIL, :], (((1,), (1,)), ((), ())),
            preferred_element_type=jnp.float32,
        )
        + b_ref[:, : _TAIL]
    )
    for s in range(_SPLIT):
      pltpu.make_async_copy(
          tacc.at[pl.ds(s * _RB, _RB), :],
          out_hbm.at[pl.ds(s * _RB, _RB), pl.ds(_N_FULL * _V_BLK, _TAIL)],
          tsems.at[s],
      ).start()
    for k in range(_NBUF):
      @pl.when((i - 1) % _NBUF == k)
      def _():
        for s in range(_SPLIT):
          _full_copy(out_hbm, acc, sems, i - 1, k, s).wait()
      @pl.when((i - 2) % _NBUF == k)
      def _():
        for s in range(_SPLIT):
          _full_copy(out_hbm, acc, sems, i - 2, k, s).wait()
    for s in range(_SPLIT):
      pltpu.make_async_copy(
          tacc.at[pl.ds(s * _RB, _RB), :],
          out_hbm.at[pl.ds(s * _RB, _RB), pl.ds(_N_FULL * _V_BLK, _TAIL)],
          tsems.at[s],
      ).wait()


@jax.jit
def kernel(inputs, embeddings, W, b):
  embeds = _gather(embeddings, inputs.astype(jnp.int32))
  out = pl.pallas_call(
      _matmul_body,
      grid=(_N_BLK,),
      in_specs=[
          pl.BlockSpec((BATCH, EMBED_DIM), lambda i: (0, 0)),
          pl.BlockSpec((_V_BLK, EMBED_DIM), lambda i: (i, 0)),
          pl.BlockSpec((1, _V_BLK), lambda i: (0, i)),
      ],
      out_specs=pl.BlockSpec(memory_space=pl.ANY),
      out_shape=jax.ShapeDtypeStruct((BATCH, VOCAB), jnp.float32),
      scratch_shapes=[
          pltpu.VMEM((_NBUF, BATCH, _V_BLK), jnp.float32),
          pltpu.VMEM((BATCH, _TAIL), jnp.float32),
          pltpu.SemaphoreType.DMA((_NBUF, _SPLIT)),
          pltpu.SemaphoreType.DMA((_SPLIT,)),
      ],
      compiler_params=pltpu.CompilerParams(
          vmem_limit_bytes=60000 * 1024,
      ),
  )(embeds, W, b.reshape(1, VOCAB))
  return out


# speedup vs baseline: 3.1777x; 2.8057x over previous
import functools

import jax
import jax.numpy as jnp
from jax import lax
from jax.experimental import pallas as pl
from jax.experimental.pallas import tpu as pltpu
from jax.experimental.pallas import tpu_sc as plsc

VOCAB = 100000
EMBED_DIM = 64
BATCH = 1024

_SC_INFO = plsc.get_sparse_core_info()
_NC = _SC_INFO.num_cores
_NS = _SC_INFO.num_subcores
_NW = _NC * _NS
_B_PER_W = BATCH // _NW

_V_BLK = 2048


def _make_gather():
  mesh = plsc.VectorSubcoreMesh(core_axis_name="c", subcore_axis_name="s")

  @functools.partial(
      pl.kernel,
      mesh=mesh,
      out_type=jax.ShapeDtypeStruct((BATCH, EMBED_DIM), jnp.float32),
      scratch_types=[
          pltpu.VMEM((_B_PER_W,), jnp.int32),
          pltpu.VMEM((_B_PER_W, EMBED_DIM), jnp.float32),
          pltpu.SemaphoreType.DMA,
      ],
      compiler_params=pltpu.CompilerParams(use_tc_tiling_on_sc=False),
  )
  def gather_kernel(table_hbm, idx_hbm, out_hbm, idx_v, rows_v, sem):
    wid = lax.axis_index("s") * _NC + lax.axis_index("c")
    base = wid * _B_PER_W
    pltpu.sync_copy(idx_hbm.at[pl.ds(base, _B_PER_W)], idx_v)
    pltpu.async_copy(table_hbm.at[idx_v], rows_v, sem).wait()
    pltpu.sync_copy(rows_v, out_hbm.at[pl.ds(base, _B_PER_W)])

  return gather_kernel


_gather = _make_gather()


def _matmul_body(w_ref, e_ref, b_ref, o_ref):
  o_ref[...] = (
      lax.dot_general(
          w_ref[...],
          e_ref[...],
          (((0,), (0,)), ((), ())),
          preferred_element_type=jnp.float32,
      )
      + b_ref[...][:, None]
  )


@jax.jit
def kernel(inputs, embeddings, W, b):
  embeds = _gather(embeddings, inputs.astype(jnp.int32))
  n_blk = pl.cdiv(VOCAB, _V_BLK)
  outT = pl.pallas_call(
      _matmul_body,
      grid=(n_blk,),
      in_specs=[
          pl.BlockSpec((EMBED_DIM, _V_BLK), lambda i: (0, i)),
          pl.BlockSpec((EMBED_DIM, BATCH), lambda i: (0, 0)),
          pl.BlockSpec((_V_BLK,), lambda i: (i,)),
      ],
      out_specs=pl.BlockSpec((_V_BLK, BATCH), lambda i: (i, 0)),
      out_shape=jax.ShapeDtypeStruct((VOCAB, BATCH), jnp.float32),
      compiler_params=pltpu.CompilerParams(
          vmem_limit_bytes=60000 * 1024,
      ),
  )(W.T, embeds.T, b)
  return outT.T


# speedup vs baseline: 4.0665x; 1.2797x over previous
import functools

import jax
import jax.numpy as jnp
from jax import lax
from jax.experimental import pallas as pl
from jax.experimental.pallas import tpu as pltpu
from jax.experimental.pallas import tpu_sc as plsc

VOCAB = 100000
EMBED_DIM = 64
BATCH = 1024

_SC_INFO = plsc.get_sparse_core_info()
_NC = _SC_INFO.num_cores
_NS = _SC_INFO.num_subcores
_NW = _NC * _NS
_B_PER_W = BATCH // _NW
_LANES = 16

_V_BLK = 2048

_TILE_W = 128
_RS = 4
_NRB = 2
_N_ROUNDS = _B_PER_W // _RS


def _make_gather():
  mesh = plsc.VectorSubcoreMesh(core_axis_name="c", subcore_axis_name="s")

  @functools.partial(
      pl.kernel,
      mesh=mesh,
      out_type=jax.ShapeDtypeStruct((BATCH, EMBED_DIM), jnp.float32),
      scratch_types=[
          pltpu.VMEM((_B_PER_W,), jnp.int32),
          pltpu.VMEM((_NRB, _RS, EMBED_DIM, _TILE_W), jnp.float32),
          pltpu.VMEM((_B_PER_W, EMBED_DIM), jnp.float32),
          pltpu.SemaphoreType.DMA,
      ],
      compiler_params=pltpu.CompilerParams(
          use_tc_tiling_on_sc=True, needs_layout_passes=False
      ),
  )
  def gather_kernel(tab_hbm, idx_hbm, out_hbm, idx_v, strips_v, rows_v, sem):
    wid = lax.axis_index("s") * _NC + lax.axis_index("c")
    base = wid * _B_PER_W
    pltpu.sync_copy(idx_hbm.at[pl.ds(base, _B_PER_W)], idx_v)

    lane_iota = lax.iota(jnp.int32, _LANES)

    def fire(r):
      buf = r % _NRB
      for q in range(_RS):
        s = r * _RS + q
        iv = idx_v[pl.ds((s // _LANES) * _LANES, _LANES)]
        c = jnp.max(jnp.where(lane_iota == (s % _LANES), iv // _TILE_W, 0))
        off = pl.multiple_of(c * _TILE_W, _TILE_W)
        pltpu.make_async_copy(
            tab_hbm.at[pl.ds(0, EMBED_DIM), pl.ds(off, _TILE_W)],
            strips_v.at[buf, q],
            sem,
        ).start()

    def drain_extract(r):
      buf = r % _NRB
      for q in range(_RS):
        s = r * _RS + q
        pltpu.make_async_copy(
            tab_hbm.at[pl.ds(0, EMBED_DIM), pl.ds(0, _TILE_W)],
            strips_v.at[buf, q],
            sem,
        ).wait()
        iv = idx_v[pl.ds((s // _LANES) * _LANES, _LANES)]
        lo = jnp.max(jnp.where(lane_iota == (s % _LANES), iv % _TILE_W, 0))
        lo_splat = jnp.full((_LANES,), lo, jnp.int32)
        for k in range(EMBED_DIM // _LANES):
          d_v = lane_iota + k * _LANES
          vals = plsc.load_gather(strips_v.at[buf, q], [d_v, lo_splat])
          rows_v[s, pl.ds(k * _LANES, _LANES)] = vals

    fire(0)
    for r in range(_N_ROUNDS):
      if r + 1 < _N_ROUNDS:
        fire(r + 1)
      drain_extract(r)

    pltpu.sync_copy(rows_v, out_hbm.at[pl.ds(base, _B_PER_W)])

  return gather_kernel


_gather = _make_gather()


def _matmul_body(w_ref, e_ref, b_ref, o_ref):
  o_ref[...] = (
      lax.dot_general(
          w_ref[...],
          e_ref[...],
          (((0,), (0,)), ((), ())),
          preferred_element_type=jnp.float32,
      )
      + b_ref[...][:, None]
  )


@jax.jit
def kernel(inputs, embeddings, W, b):
  embeds = _gather(embeddings.T, inputs.astype(jnp.int32))
  n_blk = pl.cdiv(VOCAB, _V_BLK)
  outT = pl.pallas_call(
      _matmul_body,
      grid=(n_blk,),
      in_specs=[
          pl.BlockSpec((EMBED_DIM, _V_BLK), lambda i: (0, i)),
          pl.BlockSpec((EMBED_DIM, BATCH), lambda i: (0, 0)),
          pl.BlockSpec((_V_BLK,), lambda i: (i,)),
      ],
      out_specs=pl.BlockSpec((_V_BLK, BATCH), lambda i: (i, 0)),
      out_shape=jax.ShapeDtypeStruct((VOCAB, BATCH), jnp.float32),
      compiler_params=pltpu.CompilerParams(
          vmem_limit_bytes=60000 * 1024,
      ),
  )(W.T, embeds.T, b)
  return outT.T


# speedup vs baseline: 4.1360x; 1.0171x over previous
import functools

import jax
import jax.numpy as jnp
from jax import lax
from jax.experimental import pallas as pl
from jax.experimental.pallas import tpu as pltpu
from jax.experimental.pallas import tpu_sc as plsc

VOCAB = 100000
EMBED_DIM = 64
BATCH = 1024

_SC_INFO = plsc.get_sparse_core_info()
_NC = _SC_INFO.num_cores
_NS = _SC_INFO.num_subcores
_NW = _NC * _NS
_B_PER_W = BATCH // _NW
_LANES = 16

_V_BLK = 4096

_TILE_W = 128
_RS = 4
_NRB = 2
_N_ROUNDS = _B_PER_W // _RS


def _make_gather():
  mesh = plsc.VectorSubcoreMesh(core_axis_name="c", subcore_axis_name="s")

  @functools.partial(
      pl.kernel,
      mesh=mesh,
      out_type=jax.ShapeDtypeStruct((BATCH, EMBED_DIM), jnp.float32),
      scratch_types=[
          pltpu.VMEM((_B_PER_W,), jnp.int32),
          pltpu.VMEM((_NRB, _RS, EMBED_DIM, _TILE_W), jnp.float32),
          pltpu.VMEM((_B_PER_W, EMBED_DIM), jnp.float32),
          pltpu.SemaphoreType.DMA,
      ],
      compiler_params=pltpu.CompilerParams(
          use_tc_tiling_on_sc=True, needs_layout_passes=False
      ),
  )
  def gather_kernel(tab_hbm, idx_hbm, out_hbm, idx_v, strips_v, rows_v, sem):
    wid = lax.axis_index("s") * _NC + lax.axis_index("c")
    base = wid * _B_PER_W
    pltpu.sync_copy(idx_hbm.at[pl.ds(base, _B_PER_W)], idx_v)

    lane_iota = lax.iota(jnp.int32, _LANES)

    def fire(r):
      buf = r % _NRB
      for q in range(_RS):
        s = r * _RS + q
        iv = idx_v[pl.ds((s // _LANES) * _LANES, _LANES)]
        c = jnp.max(jnp.where(lane_iota == (s % _LANES), iv // _TILE_W, 0))
        off = pl.multiple_of(c * _TILE_W, _TILE_W)
        pltpu.make_async_copy(
            tab_hbm.at[pl.ds(0, EMBED_DIM), pl.ds(off, _TILE_W)],
            strips_v.at[buf, q],
            sem,
        ).start()

    def drain_extract(r):
      buf = r % _NRB
      for q in range(_RS):
        s = r * _RS + q
        pltpu.make_async_copy(
            tab_hbm.at[pl.ds(0, EMBED_DIM), pl.ds(0, _TILE_W)],
            strips_v.at[buf, q],
            sem,
        ).wait()
        iv = idx_v[pl.ds((s // _LANES) * _LANES, _LANES)]
        lo = jnp.max(jnp.where(lane_iota == (s % _LANES), iv % _TILE_W, 0))
        lo_splat = jnp.full((_LANES,), lo, jnp.int32)
        for k in range(EMBED_DIM // _LANES):
          d_v = lane_iota + k * _LANES
          vals = plsc.load_gather(strips_v.at[buf, q], [d_v, lo_splat])
          rows_v[s, pl.ds(k * _LANES, _LANES)] = vals

    fire(0)
    for r in range(_N_ROUNDS):
      if r + 1 < _N_ROUNDS:
        fire(r + 1)
      drain_extract(r)

    pltpu.sync_copy(rows_v, out_hbm.at[pl.ds(base, _B_PER_W)])

  return gather_kernel


_gather = _make_gather()


def _matmul_body(w_ref, e_ref, b_ref, o_ref):
  o_ref[...] = (
      lax.dot_general(
          w_ref[...],
          e_ref[...],
          (((0,), (0,)), ((), ())),
          preferred_element_type=jnp.float32,
      )
      + b_ref[...][:, None]
  )


@jax.jit
def kernel(inputs, embeddings, W, b):
  embeds = _gather(embeddings.T, inputs.astype(jnp.int32))
  n_blk = pl.cdiv(VOCAB, _V_BLK)
  outT = pl.pallas_call(
      _matmul_body,
      grid=(n_blk,),
      in_specs=[
          pl.BlockSpec((EMBED_DIM, _V_BLK), lambda i: (0, i)),
          pl.BlockSpec((EMBED_DIM, BATCH), lambda i: (0, 0)),
          pl.BlockSpec((_V_BLK,), lambda i: (i,)),
      ],
      out_specs=pl.BlockSpec((_V_BLK, BATCH), lambda i: (i, 0)),
      out_shape=jax.ShapeDtypeStruct((VOCAB, BATCH), jnp.float32),
      compiler_params=pltpu.CompilerParams(
          vmem_limit_bytes=60000 * 1024,
      ),
  )(W.T, embeds.T, b)
  return outT.T
